# baseline (speedup 1.0000x reference)
import functools

import jax
import jax.numpy as jnp
from jax import lax
from jax.experimental import pallas as pl
from jax.experimental.pallas import tpu as pltpu
from jax.experimental.pallas import tpu_sc as plsc

_N = 256
_D = 128
_MARGIN = 1.0
_BIG = 1e9
_NW = 32
_APT = _N // _NW
_L = 16
_NCH = _N // _L


def _dist_kernel(emb_ref, d_ref):
    E = emb_ref[:]
    G = lax.dot_general(E, E, (((1,), (1,)), ((), ())),
                        preferred_element_type=jnp.float32)
    EE = E * E
    sq_col = jnp.sum(EE, axis=1, keepdims=True)
    ones_d = jnp.ones((1, _D), jnp.float32)
    sq_row = lax.dot_general(ones_d, EE, (((1,), (1,)), ((), ())),
                             preferred_element_type=jnp.float32)
    d_ref[...] = sq_col + sq_row - 2.0 * G


def _sc_body(d_hbm, t_hbm, out_hbm, t_v, d_v, b_v, av_v, fold_v, wl_v,
             out_v):
    wid = lax.axis_index("s") * 2 + lax.axis_index("c")
    base = wid * _APT

    pltpu.sync_copy(t_hbm, t_v.at[pl.ds(0, _N)])
    pltpu.sync_copy(d_hbm.at[pl.ds(base * _N, _APT * _N)], d_v)

    iota = lax.iota(jnp.int32, _L)
    zf = jnp.zeros((_L,), jnp.float32)
    tot_acc, vio_acc, cnt_acc = zf, zf, zf
    fold_v[pl.ds(_L, _L)] = zf - _BIG

    for a in range(_APT):
        i = base + a
        t_i = t_v[pl.ds(i, _L)][0]

        def scan_chunk(ch, carry, a=a, i=i, t_i=t_i):
            macc, wl, nact = carry
            choff = ch * _L
            kidx = choff + iota
            tch = t_v[pl.ds(choff, _L)]
            dch = d_v[pl.ds(a * _N + choff, _L)]
            same = tch == t_i
            b_v[pl.ds(a * _N + choff, _L)] = jnp.where(same, _BIG, dch)
            macc = macc + jnp.where(same, 0.0, 1.0)
            ap = same & (kidx > i)
            av = jnp.where(ap, dch + _MARGIN, -_BIG)
            av_v[pl.ds(choff, _L)] = av
            cur = av
            for sh in (8, 4, 2, 1):
                fold_v[pl.ds(0, _L)] = cur
                cur = jnp.maximum(cur, fold_v[pl.ds(sh, _L)])
            hit = cur[0] > 0.0
            wl = jnp.where(iota == nact, ch, wl)
            nact = nact + jnp.where(hit, 1, 0)
            return macc, wl, nact

        macc, wl, nact = lax.fori_loop(
            0, _NCH, scan_chunk,
            (zf, jnp.zeros((_L,), jnp.int32), jnp.int32(0)))
        wl_v[pl.ds(0, _L)] = wl

        def do_chunk(q, carry, a=a, macc=macc):
            tot, vio, cnt = carry
            ch = wl_v[pl.ds(q, _L)][0]
            av = av_v[pl.ds(ch * _L, _L)]
            avs = [av[l] for l in range(_L)]

            def k_loop(ch2, c2, a=a, avs=avs):
                tot2, vio2 = c2
                bch = b_v[pl.ds(a * _N + ch2 * _L, _L)]
                for l in range(_L):
                    t = avs[l] - bch
                    tot2 = tot2 + jnp.maximum(t, 0.0)
                    vio2 = vio2 + jnp.where(t > 0.0, 1.0, 0.0)
                return tot2, vio2

            tot, vio = lax.fori_loop(0, _NCH, k_loop, (tot, vio))
            for l in range(_L):
                cnt = cnt + jnp.where(avs[l] > 0.0, macc, zf)
            return tot, vio, cnt

        tot_acc, vio_acc, cnt_acc = lax.fori_loop(
            0, nact, do_chunk, (tot_acc, vio_acc, cnt_acc))

    out_v[pl.ds(0, _L)] = tot_acc
    out_v[pl.ds(_L, _L)] = vio_acc
    out_v[pl.ds(2 * _L, _L)] = cnt_acc
    out_v[pl.ds(3 * _L, _L)] = zf
    pltpu.sync_copy(out_v, out_hbm.at[wid])


def _sc_reduce(d_flat, t32):
    mesh = plsc.VectorSubcoreMesh(core_axis_name="c", subcore_axis_name="s")
    return pl.kernel(
        _sc_body,
        out_type=jax.ShapeDtypeStruct((_NW, 4 * _L), jnp.float32),
        mesh=mesh,
        scratch_types=[
            pltpu.VMEM((_N + _L,), jnp.int32),
            pltpu.VMEM((_APT * _N,), jnp.float32),
            pltpu.VMEM((_APT * _N,), jnp.float32),
            pltpu.VMEM((_N,), jnp.float32),
            pltpu.VMEM((2 * _L,), jnp.float32),
            pltpu.VMEM((2 * _L,), jnp.int32),
            pltpu.VMEM((4 * _L,), jnp.float32),
        ],
    )(d_flat, t32)


def kernel(embeddings, target):
    t32 = target.astype(jnp.int32)
    dmat = pl.pallas_call(
        _dist_kernel,
        out_shape=jax.ShapeDtypeStruct((_N, _N), jnp.float32),
    )(embeddings.astype(jnp.float32))
    parts = _sc_reduce(dmat.reshape(_N * _N), t32)
    total = jnp.sum(parts[:, 0:_L])
    viol = jnp.sum(parts[:, _L:2 * _L])
    count = jnp.sum(parts[:, 2 * _L:3 * _L])
    has = count > 0.5
    loss_sum = jnp.where(has, total, jnp.float32(1.0))
    ratio = jnp.where(has, viol / jnp.maximum(count, 1.0),
                      jnp.float32(1.0))
    return (loss_sum, ratio)
